# Initial kernel scaffold; baseline (speedup 1.0000x reference)
import jax
import jax.numpy as jnp
from jax.experimental import pallas as pl


def kernel(x):
    raise NotImplementedError("write your pallas kernel here")


# speedup vs baseline: 19.9533x; 19.9533x over previous
import functools

import jax
import jax.numpy as jnp
from jax import lax
from jax.experimental import pallas as pl
from jax.experimental.pallas import tpu as pltpu
from jax.experimental.pallas import tpu_sc as plsc

ROWS = 64
N = 32768
KSEL = 328
NC = 2
NS = 16
NW = NC * NS
ROWS_PER_W = ROWS // NW
LANES = 16
NVREG = N // LANES
NBUCKET = 256
HIST_SIZE = NBUCKET * LANES
SPANS = ((31, 23), (23, 15), (15, 7), (7, 0))
ABS_MASK = 0x7FFFFFFF


def _select_digit(wbuf, heads, lane, kcur):
    a_vec = lax.rev(plsc.cumsum(lax.rev(heads, (0,))), (0,))
    gstar = jnp.max(jnp.where(a_vec >= kcur, lane, jnp.int32(-1)))
    gstar_v = lane * 0 + gstar
    wsel = plsc.load_gather(wbuf, [gstar_v * 16 + lane])
    t_g = jnp.max(jnp.where(lane == gstar_v, heads, jnp.int32(0)))
    a_g = jnp.max(jnp.where(lane == gstar_v, a_vec, jnp.int32(0)))
    above_groups = a_g - t_g
    cge = wsel + above_groups
    pstar = jnp.max(jnp.where(cge >= kcur, lane, jnp.int32(-1)))
    pstar_v = lane * 0 + pstar
    wnext = jnp.max(jnp.where(lane == pstar_v + 1, wsel, jnp.int32(0)))
    new_k = kcur - (above_groups + wnext)
    bstar = gstar * 16 + pstar
    return bstar, new_k


def _tile_body(x_hbm, out_hbm, xv, hist, wbuf):
    c = lax.axis_index("c")
    s = lax.axis_index("s")
    wid = s * NC + c
    base_row = wid * ROWS_PER_W
    pltpu.sync_copy(x_hbm.at[pl.ds(base_row, ROWS_PER_W)], xv)

    lane = lax.iota(jnp.int32, LANES)
    lane_base = lane * NBUCKET
    ones = jnp.ones((LANES,), jnp.int32)
    zeros16 = jnp.zeros((LANES,), jnp.int32)

    @plsc.parallel_loop(0, HIST_SIZE // LANES, unroll=8)
    def _(j):
        hist[pl.ds(j * LANES, LANES)] = zeros16

    for r in range(ROWS_PER_W):
        kcur = jnp.int32(KSEL)
        prefix = jnp.int32(0)
        for p, (hi, lo) in enumerate(SPANS):
            if p == 0:

                @plsc.parallel_loop(0, NVREG, unroll=16)
                def _(i):
                    xv16 = xv[r, pl.ds(i * LANES, LANES)]
                    u = lax.bitcast_convert_type(xv16, jnp.int32) & ABS_MASK
                    d = u >> lo
                    plsc.addupdate_scatter(hist, [lane_base + d], ones)

            else:
                base = prefix << hi
                lim = jnp.uint32(1 << hi)

                @plsc.parallel_loop(0, NVREG, unroll=16)
                def _(i, base=base, lim=lim, lo=lo):
                    xv16 = xv[r, pl.ds(i * LANES, LANES)]
                    u = lax.bitcast_convert_type(xv16, jnp.int32) & ABS_MASK
                    t = u - base
                    m = lax.bitcast_convert_type(t, jnp.uint32) < lim
                    d = (t >> lo) & jnp.int32(0xFF)
                    plsc.addupdate_scatter(hist, [lane_base + d], ones, mask=m)

            @plsc.parallel_loop(0, NBUCKET // 16, unroll=2)
            def _(g):
                acc = zeros16
                for l in range(LANES):
                    acc = acc + hist[pl.ds(l * NBUCKET + g * 16, 16)]
                    hist[pl.ds(l * NBUCKET + g * 16, 16)] = zeros16
                w = lax.rev(plsc.cumsum(lax.rev(acc, (0,))), (0,))
                wbuf[pl.ds(g * 16, 16)] = w

            heads = plsc.load_gather(wbuf, [lane * 16])
            bstar, kcur = _select_digit(wbuf, heads, lane, kcur)
            prefix = (prefix << (hi - lo)) | bstar

        tbits = prefix

        @plsc.parallel_loop(0, NVREG, unroll=16)
        def _(i):
            xv16 = xv[r, pl.ds(i * LANES, LANES)]
            u = lax.bitcast_convert_type(xv16, jnp.int32) & ABS_MASK
            xv[r, pl.ds(i * LANES, LANES)] = jnp.where(u >= tbits, xv16, 0.0)

    pltpu.sync_copy(xv, out_hbm.at[pl.ds(base_row, ROWS_PER_W)])


@functools.partial(jax.jit, static_argnames=())
def kernel(x):
    mesh = plsc.VectorSubcoreMesh(
        core_axis_name="c", subcore_axis_name="s", num_cores=NC, num_subcores=NS
    )
    run = pl.kernel(
        _tile_body,
        out_type=jax.ShapeDtypeStruct((ROWS, N), jnp.float32),
        mesh=mesh,
        scratch_types=[
            pltpu.VMEM((ROWS_PER_W, N), jnp.float32),
            pltpu.VMEM((HIST_SIZE,), jnp.int32),
            pltpu.VMEM((NBUCKET,), jnp.int32),
        ],
        compiler_params=pltpu.CompilerParams(needs_layout_passes=False),
    )
    return run(x)


# speedup vs baseline: 20.1657x; 1.0106x over previous
import functools

import jax
import jax.numpy as jnp
from jax import lax
from jax.experimental import pallas as pl
from jax.experimental.pallas import tpu as pltpu
from jax.experimental.pallas import tpu_sc as plsc

ROWS = 64
N = 32768
KSEL = 328
NC = 2
NS = 16
NW = NC * NS
ROWS_PER_W = ROWS // NW
LANES = 16
NVREG = N // LANES
NBUCKET = 256
HIST_SIZE = NBUCKET * LANES
SPANS = ((31, 23), (23, 15), (15, 7), (7, 0))
ABS_MASK = 0x7FFFFFFF


def _select_digit(wbuf, heads, lane, kcur):
    a_vec = lax.rev(plsc.cumsum(lax.rev(heads, (0,))), (0,))
    gstar = jnp.max(jnp.where(a_vec >= kcur, lane, jnp.int32(-1)))
    gstar_v = lane * 0 + gstar
    wsel = plsc.load_gather(wbuf, [gstar_v * 16 + lane])
    t_g = jnp.max(jnp.where(lane == gstar_v, heads, jnp.int32(0)))
    a_g = jnp.max(jnp.where(lane == gstar_v, a_vec, jnp.int32(0)))
    above_groups = a_g - t_g
    cge = wsel + above_groups
    pstar = jnp.max(jnp.where(cge >= kcur, lane, jnp.int32(-1)))
    pstar_v = lane * 0 + pstar
    wnext = jnp.max(jnp.where(lane == pstar_v + 1, wsel, jnp.int32(0)))
    new_k = kcur - (above_groups + wnext)
    bstar = gstar * 16 + pstar
    return bstar, new_k


def _tile_body(x_hbm, out_hbm, xv, hist, wbuf, sem_in0, sem_in1, sem_out0, sem_out1):
    c = lax.axis_index("c")
    s = lax.axis_index("s")
    wid = s * NC + c
    base_row = wid * ROWS_PER_W
    sems_in = (sem_in0, sem_in1)
    sems_out = (sem_out0, sem_out1)
    in_copies = [
        pltpu.async_copy(x_hbm.at[base_row + r], xv.at[r], sems_in[r])
        for r in range(ROWS_PER_W)
    ]
    out_copies = []

    lane = lax.iota(jnp.int32, LANES)
    lane_base = lane * NBUCKET
    ones = jnp.ones((LANES,), jnp.int32)
    zeros16 = jnp.zeros((LANES,), jnp.int32)

    @plsc.parallel_loop(0, HIST_SIZE // LANES, unroll=8)
    def _(j):
        hist[pl.ds(j * LANES, LANES)] = zeros16

    for r in range(ROWS_PER_W):
        in_copies[r].wait()
        kcur = jnp.int32(KSEL)
        prefix = jnp.int32(0)
        for p, (hi, lo) in enumerate(SPANS):
            if p == 0:

                @plsc.parallel_loop(0, NVREG, unroll=16)
                def _(i):
                    xv16 = xv[r, pl.ds(i * LANES, LANES)]
                    u = lax.bitcast_convert_type(xv16, jnp.int32) & ABS_MASK
                    d = u >> lo
                    plsc.addupdate_scatter(hist, [lane_base + d], ones)

            else:
                base = prefix << hi
                lim = jnp.uint32(1 << hi)

                @plsc.parallel_loop(0, NVREG, unroll=16)
                def _(i, base=base, lim=lim, lo=lo):
                    xv16 = xv[r, pl.ds(i * LANES, LANES)]
                    u = lax.bitcast_convert_type(xv16, jnp.int32) & ABS_MASK
                    t = u - base
                    m = lax.bitcast_convert_type(t, jnp.uint32) < lim
                    d = (t >> lo) & jnp.int32(0xFF)
                    plsc.addupdate_scatter(hist, [lane_base + d], ones, mask=m)

            @plsc.parallel_loop(0, NBUCKET // 16, unroll=2)
            def _(g):
                acc = zeros16
                for l in range(LANES):
                    acc = acc + hist[pl.ds(l * NBUCKET + g * 16, 16)]
                    hist[pl.ds(l * NBUCKET + g * 16, 16)] = zeros16
                w = lax.rev(plsc.cumsum(lax.rev(acc, (0,))), (0,))
                wbuf[pl.ds(g * 16, 16)] = w

            heads = plsc.load_gather(wbuf, [lane * 16])
            bstar, kcur = _select_digit(wbuf, heads, lane, kcur)
            prefix = (prefix << (hi - lo)) | bstar

        tbits = prefix

        @plsc.parallel_loop(0, NVREG, unroll=16)
        def _(i):
            xv16 = xv[r, pl.ds(i * LANES, LANES)]
            u = lax.bitcast_convert_type(xv16, jnp.int32) & ABS_MASK
            xv[r, pl.ds(i * LANES, LANES)] = jnp.where(u >= tbits, xv16, 0.0)

        out_copies.append(
            pltpu.async_copy(xv.at[r], out_hbm.at[base_row + r], sems_out[r])
        )

    for cp in out_copies:
        cp.wait()


@functools.partial(jax.jit, static_argnames=())
def kernel(x):
    mesh = plsc.VectorSubcoreMesh(
        core_axis_name="c", subcore_axis_name="s", num_cores=NC, num_subcores=NS
    )
    run = pl.kernel(
        _tile_body,
        out_type=jax.ShapeDtypeStruct((ROWS, N), jnp.float32),
        mesh=mesh,
        scratch_types=[
            pltpu.VMEM((ROWS_PER_W, N), jnp.float32),
            pltpu.VMEM((HIST_SIZE,), jnp.int32),
            pltpu.VMEM((NBUCKET,), jnp.int32),
            pltpu.SemaphoreType.DMA,
            pltpu.SemaphoreType.DMA,
            pltpu.SemaphoreType.DMA,
            pltpu.SemaphoreType.DMA,
        ],
        compiler_params=pltpu.CompilerParams(needs_layout_passes=False),
    )
    return run(x)


# speedup vs baseline: 20.2170x; 1.0025x over previous
import functools

import jax
import jax.numpy as jnp
from jax import lax
from jax.experimental import pallas as pl
from jax.experimental.pallas import tpu as pltpu
from jax.experimental.pallas import tpu_sc as plsc

ROWS = 64
N = 32768
KSEL = 328
NC = 2
NS = 16
NW = NC * NS
ROWS_PER_W = ROWS // NW
LANES = 16
NVREG = N // LANES
NBUCKET = 256
HIST_SIZE = NBUCKET * LANES
SPANS = ((31, 23), (23, 15), (15, 7), (7, 0))
ABS_MASK = 0x7FFFFFFF


def _select_digit(wbuf, heads, lane, kcur):
    a_vec = lax.rev(plsc.cumsum(lax.rev(heads, (0,))), (0,))
    gstar = jnp.max(jnp.where(a_vec >= kcur, lane, jnp.int32(-1)))
    gstar_v = lane * 0 + gstar
    wsel = plsc.load_gather(wbuf, [gstar_v * 16 + lane])
    t_g = jnp.max(jnp.where(lane == gstar_v, heads, jnp.int32(0)))
    a_g = jnp.max(jnp.where(lane == gstar_v, a_vec, jnp.int32(0)))
    above_groups = a_g - t_g
    cge = wsel + above_groups
    pstar = jnp.max(jnp.where(cge >= kcur, lane, jnp.int32(-1)))
    pstar_v = lane * 0 + pstar
    wnext = jnp.max(jnp.where(lane == pstar_v + 1, wsel, jnp.int32(0)))
    new_k = kcur - (above_groups + wnext)
    bstar = gstar * 16 + pstar
    return bstar, new_k


def _tile_body(x_hbm, out_hbm, xv, hist, wbuf, sem_in0, sem_in1, sem_out0, sem_out1):
    c = lax.axis_index("c")
    s = lax.axis_index("s")
    wid = s * NC + c
    base_row = wid * ROWS_PER_W
    sems_in = (sem_in0, sem_in1)
    sems_out = (sem_out0, sem_out1)
    in_copies = [
        pltpu.async_copy(x_hbm.at[base_row + r], xv.at[r], sems_in[r])
        for r in range(ROWS_PER_W)
    ]

    lane = lax.iota(jnp.int32, LANES)
    lane_base = lane * NBUCKET
    ones = jnp.ones((LANES,), jnp.int32)
    zeros16 = jnp.zeros((LANES,), jnp.int32)

    @plsc.parallel_loop(0, ROWS_PER_W * HIST_SIZE // LANES, unroll=8)
    def _(j):
        hist[pl.ds(j * LANES, LANES)] = zeros16

    def lane_reduce_and_select(row, kcur):
        @plsc.parallel_loop(0, NBUCKET // 16, unroll=2)
        def _(g):
            acc = zeros16
            for l in range(LANES):
                sl = pl.ds(row * HIST_SIZE + l * NBUCKET + g * 16, 16)
                acc = acc + hist[sl]
                hist[sl] = zeros16
            w = lax.rev(plsc.cumsum(lax.rev(acc, (0,))), (0,))
            wbuf[pl.ds(g * 16, 16)] = w

        heads = plsc.load_gather(wbuf, [lane * 16])
        return _select_digit(wbuf, heads, lane, kcur)

    kcur = [jnp.int32(KSEL)] * ROWS_PER_W
    prefix = [jnp.int32(0)] * ROWS_PER_W
    hi1, lo1 = SPANS[0]
    for r in range(ROWS_PER_W):
        in_copies[r].wait()

        @plsc.parallel_loop(0, NVREG, unroll=16)
        def _(i, r=r):
            xv16 = xv[r, pl.ds(i * LANES, LANES)]
            u = lax.bitcast_convert_type(xv16, jnp.int32) & ABS_MASK
            d = u >> lo1
            plsc.addupdate_scatter(
                hist, [r * HIST_SIZE + lane_base + d], ones
            )

        bstar, kcur[r] = lane_reduce_and_select(r, kcur[r])
        prefix[r] = (prefix[r] << (hi1 - lo1)) | bstar

    for hi, lo in SPANS[1:]:
        bases = [prefix[r] << hi for r in range(ROWS_PER_W)]
        lim = jnp.uint32(1 << hi)

        @plsc.parallel_loop(0, NVREG, unroll=8)
        def _(i, bases=bases, lim=lim, lo=lo):
            for r in range(ROWS_PER_W):
                xv16 = xv[r, pl.ds(i * LANES, LANES)]
                u = lax.bitcast_convert_type(xv16, jnp.int32) & ABS_MASK
                t = u - bases[r]
                m = lax.bitcast_convert_type(t, jnp.uint32) < lim
                d = (t >> lo) & jnp.int32(0xFF)
                plsc.addupdate_scatter(
                    hist, [r * HIST_SIZE + lane_base + d], ones, mask=m
                )

        for r in range(ROWS_PER_W):
            bstar, kcur[r] = lane_reduce_and_select(r, kcur[r])
            prefix[r] = (prefix[r] << (hi - lo)) | bstar

    out_copies = []
    for r in range(ROWS_PER_W):
        tbits = prefix[r]

        @plsc.parallel_loop(0, NVREG, unroll=16)
        def _(i, r=r, tbits=tbits):
            xv16 = xv[r, pl.ds(i * LANES, LANES)]
            u = lax.bitcast_convert_type(xv16, jnp.int32) & ABS_MASK
            xv[r, pl.ds(i * LANES, LANES)] = jnp.where(u >= tbits, xv16, 0.0)

        out_copies.append(
            pltpu.async_copy(xv.at[r], out_hbm.at[base_row + r], sems_out[r])
        )

    for cp in out_copies:
        cp.wait()


@functools.partial(jax.jit, static_argnames=())
def kernel(x):
    mesh = plsc.VectorSubcoreMesh(
        core_axis_name="c", subcore_axis_name="s", num_cores=NC, num_subcores=NS
    )
    run = pl.kernel(
        _tile_body,
        out_type=jax.ShapeDtypeStruct((ROWS, N), jnp.float32),
        mesh=mesh,
        scratch_types=[
            pltpu.VMEM((ROWS_PER_W, N), jnp.float32),
            pltpu.VMEM((ROWS_PER_W * HIST_SIZE,), jnp.int32),
            pltpu.VMEM((NBUCKET,), jnp.int32),
            pltpu.SemaphoreType.DMA,
            pltpu.SemaphoreType.DMA,
            pltpu.SemaphoreType.DMA,
            pltpu.SemaphoreType.DMA,
        ],
        compiler_params=pltpu.CompilerParams(needs_layout_passes=False),
    )
    return run(x)


# speedup vs baseline: 21.1763x; 1.0474x over previous
import functools

import jax
import jax.numpy as jnp
from jax import lax
from jax.experimental import pallas as pl
from jax.experimental.pallas import tpu as pltpu
from jax.experimental.pallas import tpu_sc as plsc

ROWS = 64
N = 32768
KSEL = 328
NC = 2
NS = 16
NW = NC * NS
ROWS_PER_W = ROWS // NW
LANES = 16
NVREG = N // LANES
NBUCKET = 256
HIST_SIZE = NBUCKET
SPANS = ((31, 23), (23, 15), (15, 7), (7, 0))
ABS_MASK = 0x7FFFFFFF


def _select_digit(wbuf, heads, lane, kcur):
    a_vec = lax.rev(plsc.cumsum(lax.rev(heads, (0,))), (0,))
    gstar = jnp.max(jnp.where(a_vec >= kcur, lane, jnp.int32(-1)))
    gstar_v = lane * 0 + gstar
    wsel = plsc.load_gather(wbuf, [gstar_v * 16 + lane])
    t_g = jnp.max(jnp.where(lane == gstar_v, heads, jnp.int32(0)))
    a_g = jnp.max(jnp.where(lane == gstar_v, a_vec, jnp.int32(0)))
    above_groups = a_g - t_g
    cge = wsel + above_groups
    pstar = jnp.max(jnp.where(cge >= kcur, lane, jnp.int32(-1)))
    pstar_v = lane * 0 + pstar
    wnext = jnp.max(jnp.where(lane == pstar_v + 1, wsel, jnp.int32(0)))
    new_k = kcur - (above_groups + wnext)
    bstar = gstar * 16 + pstar
    return bstar, new_k


def _tile_body(x_hbm, out_hbm, xv, hist, wbuf, sem_in0, sem_in1, sem_out0, sem_out1):
    c = lax.axis_index("c")
    s = lax.axis_index("s")
    wid = s * NC + c
    base_row = wid * ROWS_PER_W
    sems_in = (sem_in0, sem_in1)
    sems_out = (sem_out0, sem_out1)
    in_copies = [
        pltpu.async_copy(x_hbm.at[base_row + r], xv.at[r], sems_in[r])
        for r in range(ROWS_PER_W)
    ]
    out_copies = []

    lane = lax.iota(jnp.int32, LANES)
    ones = jnp.ones((LANES,), jnp.int32)
    zeros16 = jnp.zeros((LANES,), jnp.int32)

    @plsc.parallel_loop(0, HIST_SIZE // LANES, unroll=8)
    def _(j):
        hist[pl.ds(j * LANES, LANES)] = zeros16

    for r in range(ROWS_PER_W):
        in_copies[r].wait()
        kcur = jnp.int32(KSEL)
        prefix = jnp.int32(0)
        for p, (hi, lo) in enumerate(SPANS):
            if p == 0:

                @plsc.parallel_loop(0, NVREG, unroll=16)
                def _(i):
                    xv16 = xv[r, pl.ds(i * LANES, LANES)]
                    u = lax.bitcast_convert_type(xv16, jnp.int32) & ABS_MASK
                    d = u >> lo
                    plsc.addupdate_scatter(hist, [d], ones)

            else:
                base = prefix << hi
                lim = jnp.uint32(1 << hi)

                @plsc.parallel_loop(0, NVREG, unroll=16)
                def _(i, base=base, lim=lim, lo=lo):
                    xv16 = xv[r, pl.ds(i * LANES, LANES)]
                    u = lax.bitcast_convert_type(xv16, jnp.int32) & ABS_MASK
                    t = u - base
                    m = lax.bitcast_convert_type(t, jnp.uint32) < lim
                    d = (t >> lo) & jnp.int32(0xFF)
                    plsc.addupdate_scatter(hist, [d], ones, mask=m)

            @plsc.parallel_loop(0, NBUCKET // 16, unroll=2)
            def _(g):
                acc = hist[pl.ds(g * 16, 16)]
                hist[pl.ds(g * 16, 16)] = zeros16
                w = lax.rev(plsc.cumsum(lax.rev(acc, (0,))), (0,))
                wbuf[pl.ds(g * 16, 16)] = w

            heads = plsc.load_gather(wbuf, [lane * 16])
            bstar, kcur = _select_digit(wbuf, heads, lane, kcur)
            prefix = (prefix << (hi - lo)) | bstar

        tbits = prefix

        @plsc.parallel_loop(0, NVREG, unroll=16)
        def _(i):
            xv16 = xv[r, pl.ds(i * LANES, LANES)]
            u = lax.bitcast_convert_type(xv16, jnp.int32) & ABS_MASK
            xv[r, pl.ds(i * LANES, LANES)] = jnp.where(u >= tbits, xv16, 0.0)

        out_copies.append(
            pltpu.async_copy(xv.at[r], out_hbm.at[base_row + r], sems_out[r])
        )

    for cp in out_copies:
        cp.wait()


@functools.partial(jax.jit, static_argnames=())
def kernel(x):
    mesh = plsc.VectorSubcoreMesh(
        core_axis_name="c", subcore_axis_name="s", num_cores=NC, num_subcores=NS
    )
    run = pl.kernel(
        _tile_body,
        out_type=jax.ShapeDtypeStruct((ROWS, N), jnp.float32),
        mesh=mesh,
        scratch_types=[
            pltpu.VMEM((ROWS_PER_W, N), jnp.float32),
            pltpu.VMEM((HIST_SIZE,), jnp.int32),
            pltpu.VMEM((NBUCKET,), jnp.int32),
            pltpu.SemaphoreType.DMA,
            pltpu.SemaphoreType.DMA,
            pltpu.SemaphoreType.DMA,
            pltpu.SemaphoreType.DMA,
        ],
        compiler_params=pltpu.CompilerParams(needs_layout_passes=False),
    )
    return run(x)


# speedup vs baseline: 27.1624x; 1.2827x over previous
import functools

import jax
import jax.numpy as jnp
from jax import lax
from jax.experimental import pallas as pl
from jax.experimental.pallas import tpu as pltpu
from jax.experimental.pallas import tpu_sc as plsc

ROWS = 64
N = 32768
KSEL = 328
NC = 2
NS = 16
NW = NC * NS
ROWS_PER_W = ROWS // NW
LANES = 16
NVREG = N // LANES
NB1 = 2048
NB23 = 1024
SPANS = ((31, 20), (20, 10), (10, 0))
ABS_MASK = 0x7FFFFFFF


def _pick(cge, kcur, lane):
    istar = jnp.max(jnp.where(cge >= kcur, lane, jnp.int32(-1)))
    istar_v = lane * 0 + istar
    nxt = jnp.max(jnp.where(lane == istar_v + 1, cge, jnp.int32(0)))
    return istar, istar_v, nxt


def _tile_body(x_hbm, out_hbm, xv, hist, wbuf, s2buf, sem_in0, sem_in1,
               sem_out0, sem_out1):
    c = lax.axis_index("c")
    s = lax.axis_index("s")
    wid = s * NC + c
    base_row = wid * ROWS_PER_W
    sems_in = (sem_in0, sem_in1)
    sems_out = (sem_out0, sem_out1)
    in_copies = [
        pltpu.async_copy(x_hbm.at[base_row + r], xv.at[r], sems_in[r])
        for r in range(ROWS_PER_W)
    ]

    lane = lax.iota(jnp.int32, LANES)
    ones = jnp.ones((LANES,), jnp.int32)
    zeros16 = jnp.zeros((LANES,), jnp.int32)

    @plsc.parallel_loop(0, NB1 // LANES, unroll=8)
    def _(j):
        hist[pl.ds(j * LANES, LANES)] = zeros16

    @plsc.parallel_loop(0, 256 // LANES)
    def _(j):
        s2buf[pl.ds(j * LANES, LANES)] = zeros16

    def select(nb, kcur):
        ngroup = nb // 16
        nsuper = ngroup // 16

        @plsc.parallel_loop(0, ngroup, unroll=4)
        def _(g):
            acc = hist[pl.ds(g * 16, 16)]
            hist[pl.ds(g * 16, 16)] = zeros16
            w = lax.rev(plsc.cumsum(lax.rev(acc, (0,))), (0,))
            wbuf[pl.ds(g * 16, 16)] = w

        for h in range(nsuper):
            heads_h = plsc.load_gather(wbuf, [(h * 16 + lane) * 16])
            s2 = lax.rev(plsc.cumsum(lax.rev(heads_h, (0,))), (0,))
            s2buf[pl.ds(h * 16, 16)] = s2

        tops = plsc.load_gather(s2buf, [lane * 16])
        a3 = lax.rev(plsc.cumsum(lax.rev(tops, (0,))), (0,))

        hstar, hstar_v, next3 = _pick(a3, kcur, lane)
        above3 = next3

        s2sel = plsc.load_gather(s2buf, [hstar_v * 16 + lane])
        cge2 = s2sel + above3
        gloc, gloc_v, next2 = _pick(cge2, kcur, lane)
        above2 = jnp.where(gloc == 15, above3, next2)
        gstar = hstar * 16 + gloc

        gstar_v = lane * 0 + gstar
        wsel = plsc.load_gather(wbuf, [gstar_v * 16 + lane])
        cge1 = wsel + above2
        ploc, _, next1 = _pick(cge1, kcur, lane)
        above1 = jnp.where(ploc == 15, above2, next1)
        bstar = gstar * 16 + ploc
        new_k = kcur - above1
        return bstar, new_k

    out_copies = []
    for r in range(ROWS_PER_W):
        in_copies[r].wait()
        kcur = jnp.int32(KSEL)

        hi, lo = SPANS[0]

        @plsc.parallel_loop(0, NVREG, unroll=8)
        def _(i, r=r, lo=lo):
            xv16 = xv[r, pl.ds(i * LANES, LANES)]
            u = lax.bitcast_convert_type(xv16, jnp.int32) & ABS_MASK
            d = u >> lo
            plsc.addupdate_scatter(hist, [d], ones)

        bstar, kcur = select(NB1, kcur)
        prefix = bstar

        for hi, lo in SPANS[1:]:
            base = prefix << hi
            lim = jnp.uint32(1 << hi)

            @plsc.parallel_loop(0, NVREG, unroll=8)
            def _(i, r=r, base=base, lim=lim, lo=lo):
                xv16 = xv[r, pl.ds(i * LANES, LANES)]
                u = lax.bitcast_convert_type(xv16, jnp.int32) & ABS_MASK
                t = u - base
                m = lax.bitcast_convert_type(t, jnp.uint32) < lim
                d = (t >> lo) & jnp.int32(0x3FF)
                plsc.addupdate_scatter(hist, [d], ones, mask=m)

            bstar, kcur = select(NB23, kcur)
            prefix = (prefix << (hi - lo)) | bstar

        tbits = prefix

        @plsc.parallel_loop(0, NVREG, unroll=8)
        def _(i, r=r, tbits=tbits):
            xv16 = xv[r, pl.ds(i * LANES, LANES)]
            u = lax.bitcast_convert_type(xv16, jnp.int32) & ABS_MASK
            xv[r, pl.ds(i * LANES, LANES)] = jnp.where(u >= tbits, xv16, 0.0)

        out_copies.append(
            pltpu.async_copy(xv.at[r], out_hbm.at[base_row + r], sems_out[r])
        )

    for cp in out_copies:
        cp.wait()


@functools.partial(jax.jit, static_argnames=())
def kernel(x):
    mesh = plsc.VectorSubcoreMesh(
        core_axis_name="c", subcore_axis_name="s", num_cores=NC, num_subcores=NS
    )
    run = pl.kernel(
        _tile_body,
        out_type=jax.ShapeDtypeStruct((ROWS, N), jnp.float32),
        mesh=mesh,
        scratch_types=[
            pltpu.VMEM((ROWS_PER_W, N), jnp.float32),
            pltpu.VMEM((NB1,), jnp.int32),
            pltpu.VMEM((NB1,), jnp.int32),
            pltpu.VMEM((256,), jnp.int32),
            pltpu.SemaphoreType.DMA,
            pltpu.SemaphoreType.DMA,
            pltpu.SemaphoreType.DMA,
            pltpu.SemaphoreType.DMA,
        ],
        compiler_params=pltpu.CompilerParams(needs_layout_passes=False),
    )
    return run(x)
